# baseline (speedup 1.0000x reference)
import jax
import jax.numpy as jnp
from jax import lax
from jax.experimental import pallas as pl
from jax.experimental.pallas import tpu as pltpu
from jax.experimental.pallas import tpu_sc as plsc

N = 10000
E = 320000
D_FEAT = 128
HID = 128
EPS = 1e-5

NC = 2
NS = 16
NW = NC * NS
EPW = E // NW
BLK = 80
NBLK = EPW // BLK
N_PAD = 10240
RPT = N_PAD // NS

ROWS = 1000
GRID = N // ROWS

_mesh_cache = []


def _mesh():
    if not _mesh_cache:
        _mesh_cache.append(plsc.VectorSubcoreMesh(
            core_axis_name="c", subcore_axis_name="s",
            num_cores=NC, num_subcores=NS))
    return _mesh_cache[0]


def _deg_body(dst_hbm, ones_hbm, z_hbm, out_hbm, ones_v, didx0, didx1,
              acc_sh, sem_a, sem_b, semi0, semi1):
    cid = lax.axis_index("c")
    sid = lax.axis_index("s")
    wid = sid * NC + cid
    pltpu.sync_copy(z_hbm, acc_sh.at[pl.ds(sid * RPT, RPT)])
    pltpu.sync_copy(ones_hbm, ones_v)
    base = wid * EPW

    def idx_issue(b, dbuf, sem):
        off = pl.multiple_of(base + b * BLK, 8)
        pltpu.async_copy(dst_hbm.at[pl.ds(off, BLK)], dbuf, sem)

    def idx_wait(b, dbuf, sem):
        off = pl.multiple_of(base + b * BLK, 8)
        pltpu.make_async_copy(dst_hbm.at[pl.ds(off, BLK)], dbuf, sem).wait()

    def scat(dbuf, sem):
        pltpu.async_copy(ones_v, acc_sh.at[dbuf], sem, add=True)

    def scat_wait(dbuf, sem):
        pltpu.make_async_copy(ones_v, acc_sh.at[dbuf], sem).wait()

    off0 = pl.multiple_of(base, 8)
    pltpu.sync_copy(dst_hbm.at[pl.ds(off0, BLK)], didx0)
    idx_issue(1, didx1, semi1)
    plsc.subcore_barrier()
    scat(didx0, sem_a)

    def body(i, carry):
        b0 = i * 2
        idx_wait(b0 + 1, didx1, semi1)
        scat_wait(didx0, sem_a)
        scat(didx1, sem_b)
        idx_issue(b0 + 2, didx0, semi0)
        idx_wait(b0 + 2, didx0, semi0)
        scat_wait(didx1, sem_b)
        scat(didx0, sem_a)
        b3 = jnp.minimum(b0 + 3, NBLK - 1)
        idx_issue(b3, didx1, semi1)
        return carry

    lax.fori_loop(0, (NBLK - 1) // 2, body, 0)
    scat_wait(didx0, sem_a)
    idx_wait(NBLK - 1, didx1, semi1)
    plsc.subcore_barrier()
    pltpu.sync_copy(
        acc_sh.at[pl.ds(sid * RPT, RPT)], out_hbm.at[cid, pl.ds(sid * RPT, RPT)]
    )


def _deg_call(dst, ones_blk, z128):
    return pl.kernel(
        _deg_body,
        out_type=jax.ShapeDtypeStruct((NC, N_PAD, HID), jnp.float32),
        mesh=_mesh(),
        scratch_types=[
            pltpu.VMEM((BLK, HID), jnp.float32),
            pltpu.VMEM((BLK,), jnp.int32),
            pltpu.VMEM((BLK,), jnp.int32),
            pltpu.VMEM_SHARED((N_PAD, HID), jnp.float32),
            pltpu.SemaphoreType.DMA,
            pltpu.SemaphoreType.DMA,
            pltpu.SemaphoreType.DMA,
            pltpu.SemaphoreType.DMA,
        ],
    )(dst, ones_blk, z128)


def _agg_body(src_hbm, dst_hbm, mt_hbm, z_hbm, out_hbm, *scr):
    sidx = scr[0:8]
    didx = scr[8:16]
    rows = scr[16:20]
    acc_sh = scr[20]
    semg = scr[21:25]
    sems = scr[25:29]
    semi = scr[29:37]
    cid = lax.axis_index("c")
    sid = lax.axis_index("s")
    wid = sid * NC + cid
    pltpu.sync_copy(z_hbm, acc_sh.at[pl.ds(sid * RPT, RPT)])
    base = wid * EPW

    def idx_issue(b, j8):
        off = pl.multiple_of(base + b * BLK, 8)
        pltpu.async_copy(src_hbm.at[pl.ds(off, BLK)], sidx[j8], semi[j8])
        pltpu.async_copy(dst_hbm.at[pl.ds(off, BLK)], didx[j8], semi[j8])

    def idx_wait(b, j8):
        off = pl.multiple_of(base + b * BLK, 8)
        pltpu.make_async_copy(src_hbm.at[pl.ds(off, BLK)], sidx[j8], semi[j8]).wait()
        pltpu.make_async_copy(dst_hbm.at[pl.ds(off, BLK)], didx[j8], semi[j8]).wait()

    def idx_sync(b, j8):
        off = pl.multiple_of(base + b * BLK, 8)
        pltpu.sync_copy(src_hbm.at[pl.ds(off, BLK)], sidx[j8])
        pltpu.sync_copy(dst_hbm.at[pl.ds(off, BLK)], didx[j8])

    def gat_issue(j8, j4):
        pltpu.async_copy(mt_hbm.at[sidx[j8]], rows[j4], semg[j4])

    def gat_wait(j8, j4):
        pltpu.make_async_copy(mt_hbm.at[sidx[j8]], rows[j4], semg[j4]).wait()

    def scat_issue(j8, j4):
        pltpu.async_copy(rows[j4], acc_sh.at[didx[j8]], sems[j4], add=True)

    def scat_wait(j8, j4):
        pltpu.make_async_copy(rows[j4], acc_sh.at[didx[j8]], sems[j4]).wait()

    def stage(b, j8, head=False, tail=0):
        j4 = j8 % 4
        if not head:
            scat_wait((j8 + 6) % 8, (j4 + 2) % 4)
        if tail < 1:
            idx_issue(b + 4, (j8 + 4) % 8)
        if tail < 2:
            idx_wait(b + 2, (j8 + 2) % 8)
            gat_issue((j8 + 2) % 8, (j4 + 2) % 4)
        gat_wait(j8, j4)
        scat_issue(j8, j4)

    idx_sync(0, 0)
    idx_sync(1, 1)
    idx_issue(2, 2)
    idx_issue(3, 3)
    gat_issue(0, 0)
    gat_issue(1, 1)
    plsc.subcore_barrier()

    stage(0, 0, head=True)
    stage(1, 1, head=True)
    for b in range(2, 8):
        stage(b, b % 8)

    def body(i, carry):
        b0 = 8 + i * 8
        for j in range(8):
            stage(b0 + j, j)
        return carry

    lax.fori_loop(0, (NBLK - 13) // 8, body, 0)

    stage(NBLK - 5, 0)
    stage(NBLK - 4, 1, tail=1)
    stage(NBLK - 3, 2, tail=1)
    stage(NBLK - 2, 3, tail=2)
    stage(NBLK - 1, 4, tail=2)
    scat_wait(3, 3)
    scat_wait(4, 0)
    plsc.subcore_barrier()
    pltpu.sync_copy(
        acc_sh.at[pl.ds(sid * RPT, RPT)], out_hbm.at[cid, pl.ds(sid * RPT, RPT)]
    )


def _agg_call(src1, dst1, mt, z128):
    idx_t = pltpu.VMEM((BLK,), jnp.int32)
    row_t = pltpu.VMEM((BLK, HID), jnp.float32)
    dma = pltpu.SemaphoreType.DMA
    return pl.kernel(
        _agg_body,
        out_type=jax.ShapeDtypeStruct((NC, N_PAD, HID), jnp.float32),
        mesh=_mesh(),
        scratch_types=(
            [idx_t] * 16 + [row_t] * 4
            + [pltpu.VMEM_SHARED((N_PAD, HID), jnp.float32)]
            + [dma] * 16
        ),
    )(src1, dst1, mt, z128)


_BNS = float(1.0 / (1.0 + EPS) ** 0.5)


def _mm_t(a, w):
    return lax.dot_general(a, w, (((1,), (1,)), ((), ())),
                           preferred_element_type=jnp.float32)


def _silu(x):
    return x * jax.nn.sigmoid(x)


def _tc0a_body(x_ref, pp_ref, wx_ref, wp_ref, bin_ref, gin_ref, bein_ref,
               h_ref):
    lin = _mm_t(x_ref[...], wx_ref[...]) + _mm_t(pp_ref[...], wp_ref[...])
    lin = lin + bin_ref[...]
    y = lin * (gin_ref[...] * _BNS) + bein_ref[...]
    h_ref[...] = _silu(y)


def _tc0a_call(x, pos_p, Wx, Wp, b_in, g_in, be_in):
    rb = lambda i: (i, 0)
    wb = lambda i: (0, 0)
    return pl.pallas_call(
        _tc0a_body,
        grid=(GRID,),
        in_specs=[
            pl.BlockSpec((ROWS, D_FEAT), rb),
            pl.BlockSpec((ROWS, 8), rb),
            pl.BlockSpec((HID, D_FEAT), wb),
            pl.BlockSpec((HID, 8), wb),
            pl.BlockSpec((1, HID), wb),
            pl.BlockSpec((1, HID), wb),
            pl.BlockSpec((1, HID), wb),
        ],
        out_specs=pl.BlockSpec((ROWS, HID), rb),
        out_shape=jax.ShapeDtypeStruct((N, HID), jnp.float32),
    )(x, pos_p, Wx, Wp, b_in, g_in, be_in)


def _tc0b_body(h_ref, dp_ref, wg0_ref, mt_ref, dinv_ref):
    deg = dp_ref[0, :, 0:1] + dp_ref[1, :, 0:1] + 1.0
    dinv = lax.rsqrt(deg)
    dinv_ref[...] = jnp.broadcast_to(dinv, (ROWS, 8))
    mt_ref[...] = _mm_t(h_ref[...], wg0_ref[...]) * dinv


def _tc0b_call(h, dp8, Wg0):
    rb = lambda i: (i, 0)
    return pl.pallas_call(
        _tc0b_body,
        grid=(GRID,),
        in_specs=[
            pl.BlockSpec((ROWS, HID), rb),
            pl.BlockSpec((NC, ROWS, 8), lambda i: (0, i, 0)),
            pl.BlockSpec((HID, HID), lambda i: (0, 0)),
        ],
        out_specs=[
            pl.BlockSpec((ROWS, HID), rb),
            pl.BlockSpec((ROWS, 8), rb),
        ],
        out_shape=[
            jax.ShapeDtypeStruct((N, HID), jnp.float32),
            jax.ShapeDtypeStruct((N, 8), jnp.float32),
        ],
    )(h, dp8, Wg0)


def _epi_body(p_ref, mt_ref, hp_ref, dinv_ref, bg_ref, gn_ref, bn_ref,
              wgn_ref, h_ref, mtn_ref):
    dinv = jnp.broadcast_to(dinv_ref[:, 0:1], (ROWS, HID))
    agg = p_ref[0] + p_ref[1]
    out = dinv * (agg + mt_ref[...]) + bg_ref[...]
    y = out * (gn_ref[...] * _BNS) + bn_ref[...]
    h = _silu(y + hp_ref[...])
    h_ref[...] = h
    mtn_ref[...] = _mm_t(h, wgn_ref[...]) * dinv


def _epi_call(p, mt, h_prev, dinv8, bg, gn, bn, Wg_next):
    rb = lambda i: (i, 0)
    wb = lambda i: (0, 0)
    return pl.pallas_call(
        _epi_body,
        grid=(GRID,),
        in_specs=[
            pl.BlockSpec((NC, ROWS, HID), lambda i: (0, i, 0)),
            pl.BlockSpec((ROWS, HID), rb),
            pl.BlockSpec((ROWS, HID), rb),
            pl.BlockSpec((ROWS, 8), rb),
            pl.BlockSpec((1, HID), wb),
            pl.BlockSpec((1, HID), wb),
            pl.BlockSpec((1, HID), wb),
            pl.BlockSpec((HID, HID), wb),
        ],
        out_specs=[
            pl.BlockSpec((ROWS, HID), rb),
            pl.BlockSpec((ROWS, HID), rb),
        ],
        out_shape=[
            jax.ShapeDtypeStruct((N, HID), jnp.float32),
            jax.ShapeDtypeStruct((N, HID), jnp.float32),
        ],
    )(p, mt, h_prev, dinv8, bg, gn, bn, Wg_next)


def _epi_pool_body(p_ref, mt_ref, hp_ref, dinv_ref, bg_ref, gn_ref, bn_ref,
                   watt_ref, wo1_ref, bo1_ref, wo2_ref, bo2_ref, wo3_ref,
                   h_ref, pred_ref, m_sc, s_sc, v_sc):
    i = pl.program_id(0)
    dinv = jnp.broadcast_to(dinv_ref[:, 0:1], (ROWS, HID))
    agg = p_ref[0] + p_ref[1]
    out = dinv * (agg + mt_ref[...]) + bg_ref[...]
    y = out * (gn_ref[...] * _BNS) + bn_ref[...]
    h = _silu(y + hp_ref[...])
    h_ref[...] = h
    wl = jnp.sum(h * watt_ref[...], axis=1, keepdims=True)
    mb = jnp.max(wl)

    @pl.when(i == 0)
    def _():
        m_sc[0, 0] = mb
        s_sc[0, 0] = 0.0
        v_sc[...] = jnp.zeros((1, HID), jnp.float32)

    m_old = m_sc[0, 0]
    m_new = jnp.maximum(m_old, mb)
    corr = jnp.exp(m_old - m_new)
    e = jnp.exp(wl - m_new)
    s_sc[0, 0] = s_sc[0, 0] * corr + jnp.sum(e)
    v_sc[...] = v_sc[...] * corr + jnp.sum(h * e, axis=0, keepdims=True)
    m_sc[0, 0] = m_new

    @pl.when(i == GRID - 1)
    def _():
        hg = v_sc[...] / s_sc[0, 0]
        o = _silu(_mm_t(hg, wo1_ref[...]) + bo1_ref[...])
        o = _silu(_mm_t(o, wo2_ref[...]) + bo2_ref[...])
        pred_ref[...] = _mm_t(o, wo3_ref[...])


def _epi_pool_call(p, mt, h_prev, dinv8, bg, gn, bn, W_att, Wo1, bo1, Wo2,
                   bo2, Wo3):
    rb = lambda i: (i, 0)
    wb = lambda i: (0, 0)
    return pl.pallas_call(
        _epi_pool_body,
        grid=(GRID,),
        in_specs=[
            pl.BlockSpec((NC, ROWS, HID), lambda i: (0, i, 0)),
            pl.BlockSpec((ROWS, HID), rb),
            pl.BlockSpec((ROWS, HID), rb),
            pl.BlockSpec((ROWS, 8), rb),
            pl.BlockSpec((1, HID), wb),
            pl.BlockSpec((1, HID), wb),
            pl.BlockSpec((1, HID), wb),
            pl.BlockSpec((1, HID), wb),
            pl.BlockSpec((HID, HID), wb),
            pl.BlockSpec((1, HID), wb),
            pl.BlockSpec((HID // 2, HID), wb),
            pl.BlockSpec((1, HID // 2), wb),
            pl.BlockSpec((1, HID // 2), wb),
        ],
        out_specs=[
            pl.BlockSpec((ROWS, HID), rb),
            pl.BlockSpec((1, 1), wb),
        ],
        out_shape=[
            jax.ShapeDtypeStruct((N, HID), jnp.float32),
            jax.ShapeDtypeStruct((1, 1), jnp.float32),
        ],
        scratch_shapes=[
            pltpu.SMEM((1, 1), jnp.float32),
            pltpu.SMEM((1, 1), jnp.float32),
            pltpu.VMEM((1, HID), jnp.float32),
        ],
    )(p, mt, h_prev, dinv8, bg, gn, bn, W_att, Wo1, bo1, Wo2, bo2, Wo3)


def kernel(x, pos, edge_index, W_in, b_in, g_in, be_in, Wg0, bg0, gn0, bn0,
           Wg1, bg1, gn1, bn1, Wg2, bg2, gn2, bn2, Wg3, bg3, gn3, bn3,
           W_att, b_att, Wo1, bo1, Wo2, bo2, Wo3, bo3):
    f32 = jnp.float32
    src1 = edge_index[0]
    dst1 = edge_index[1]
    pos_p = jnp.concatenate([pos, jnp.zeros((N, 5), f32)], axis=1)
    Wx = W_in[:, :D_FEAT]
    Wp = jnp.concatenate([W_in[:, D_FEAT:], jnp.zeros((HID, 5), f32)], axis=1)
    z128 = jnp.zeros((RPT, HID), f32)
    ones_blk = jnp.ones((BLK, HID), f32)

    dp = _deg_call(dst1, ones_blk, z128)
    h = _tc0a_call(x, pos_p, Wx, Wp, b_in.reshape(1, HID),
                   g_in.reshape(1, HID), be_in.reshape(1, HID))
    mt, dinv8 = _tc0b_call(h, dp[:, :N, :8], Wg0)

    mids = [(bg0, gn0, bn0, Wg1), (bg1, gn1, bn1, Wg2), (bg2, gn2, bn2, Wg3)]
    for bg, gn, bn, Wg_next in mids:
        p = _agg_call(src1, dst1, mt, z128)
        h, mt = _epi_call(p, mt, h, dinv8, bg.reshape(1, HID),
                          gn.reshape(1, HID), bn.reshape(1, HID), Wg_next)
    p = _agg_call(src1, dst1, mt, z128)
    h, pred = _epi_pool_call(p, mt, h, dinv8, bg3.reshape(1, HID),
                             gn3.reshape(1, HID), bn3.reshape(1, HID),
                             W_att, Wo1, bo1.reshape(1, HID), Wo2,
                             bo2.reshape(1, HID // 2), Wo3)
    pred = pred + bo3.reshape(1, 1)
    return (pred, h)


# speedup vs baseline: 1.0103x; 1.0103x over previous
import jax
import jax.numpy as jnp
from jax import lax
from jax.experimental import pallas as pl
from jax.experimental.pallas import tpu as pltpu
from jax.experimental.pallas import tpu_sc as plsc

N = 10000
E = 320000
D_FEAT = 128
HID = 128
EPS = 1e-5

NC = 2
NS = 16
NW = NC * NS
EPW = E // NW
BLK = 80
NBLK = EPW // BLK
N_PAD = 10240
RPT = N_PAD // NS

ROWS = 1000
GRID = N // ROWS

_mesh_cache = []


def _mesh():
    if not _mesh_cache:
        _mesh_cache.append(plsc.VectorSubcoreMesh(
            core_axis_name="c", subcore_axis_name="s",
            num_cores=NC, num_subcores=NS))
    return _mesh_cache[0]


def _deg_body(dst_hbm, ones_hbm, z_hbm, out_hbm, ones_v, didx0, didx1,
              acc_sh, sem_a, sem_b, semi0, semi1):
    cid = lax.axis_index("c")
    sid = lax.axis_index("s")
    wid = sid * NC + cid
    pltpu.sync_copy(z_hbm, acc_sh.at[pl.ds(sid * RPT, RPT)])
    pltpu.sync_copy(ones_hbm, ones_v)
    base = wid * EPW

    def idx_issue(b, dbuf, sem):
        off = pl.multiple_of(base + b * BLK, 8)
        pltpu.async_copy(dst_hbm.at[pl.ds(off, BLK)], dbuf, sem)

    def idx_wait(b, dbuf, sem):
        off = pl.multiple_of(base + b * BLK, 8)
        pltpu.make_async_copy(dst_hbm.at[pl.ds(off, BLK)], dbuf, sem).wait()

    def scat(dbuf, sem):
        pltpu.async_copy(ones_v, acc_sh.at[dbuf], sem, add=True)

    def scat_wait(dbuf, sem):
        pltpu.make_async_copy(ones_v, acc_sh.at[dbuf], sem).wait()

    off0 = pl.multiple_of(base, 8)
    pltpu.sync_copy(dst_hbm.at[pl.ds(off0, BLK)], didx0)
    idx_issue(1, didx1, semi1)
    plsc.subcore_barrier()
    scat(didx0, sem_a)

    def body(i, carry):
        b0 = i * 2
        idx_wait(b0 + 1, didx1, semi1)
        scat_wait(didx0, sem_a)
        scat(didx1, sem_b)
        idx_issue(b0 + 2, didx0, semi0)
        idx_wait(b0 + 2, didx0, semi0)
        scat_wait(didx1, sem_b)
        scat(didx0, sem_a)
        b3 = jnp.minimum(b0 + 3, NBLK - 1)
        idx_issue(b3, didx1, semi1)
        return carry

    lax.fori_loop(0, (NBLK - 1) // 2, body, 0)
    scat_wait(didx0, sem_a)
    idx_wait(NBLK - 1, didx1, semi1)
    plsc.subcore_barrier()
    pltpu.sync_copy(
        acc_sh.at[pl.ds(sid * RPT, RPT)], out_hbm.at[cid, pl.ds(sid * RPT, RPT)]
    )


def _deg_call(dst, ones_blk, z128):
    return pl.kernel(
        _deg_body,
        out_type=jax.ShapeDtypeStruct((NC, N_PAD, HID), jnp.float32),
        mesh=_mesh(),
        scratch_types=[
            pltpu.VMEM((BLK, HID), jnp.float32),
            pltpu.VMEM((BLK,), jnp.int32),
            pltpu.VMEM((BLK,), jnp.int32),
            pltpu.VMEM_SHARED((N_PAD, HID), jnp.float32),
            pltpu.SemaphoreType.DMA,
            pltpu.SemaphoreType.DMA,
            pltpu.SemaphoreType.DMA,
            pltpu.SemaphoreType.DMA,
        ],
    )(dst, ones_blk, z128)


def _agg_body(src_hbm, dst_hbm, mt_hbm, z_hbm, out_hbm, *scr):
    sidx = scr[0:8]
    didx = scr[8:16]
    rows = scr[16:20]
    acc_sh = scr[20]
    semg = scr[21:25]
    sems = scr[25:29]
    semi = scr[29:37]
    cid = lax.axis_index("c")
    sid = lax.axis_index("s")
    wid = sid * NC + cid
    pltpu.sync_copy(z_hbm, acc_sh.at[pl.ds(sid * RPT, RPT)])
    base = wid * EPW

    def idx_issue(b, j8):
        off = pl.multiple_of(base + b * BLK, 8)
        pltpu.async_copy(src_hbm.at[pl.ds(off, BLK)], sidx[j8], semi[j8])
        pltpu.async_copy(dst_hbm.at[pl.ds(off, BLK)], didx[j8], semi[j8])

    def idx_wait(b, j8):
        off = pl.multiple_of(base + b * BLK, 8)
        pltpu.make_async_copy(src_hbm.at[pl.ds(off, BLK)], sidx[j8], semi[j8]).wait()
        pltpu.make_async_copy(dst_hbm.at[pl.ds(off, BLK)], didx[j8], semi[j8]).wait()

    def idx_sync(b, j8):
        off = pl.multiple_of(base + b * BLK, 8)
        pltpu.sync_copy(src_hbm.at[pl.ds(off, BLK)], sidx[j8])
        pltpu.sync_copy(dst_hbm.at[pl.ds(off, BLK)], didx[j8])

    def gat_issue(j8, j4):
        pltpu.async_copy(mt_hbm.at[sidx[j8]], rows[j4], semg[j4])

    def gat_wait(j8, j4):
        pltpu.make_async_copy(mt_hbm.at[sidx[j8]], rows[j4], semg[j4]).wait()

    def scat_issue(j8, j4):
        pltpu.async_copy(rows[j4], acc_sh.at[didx[j8]], sems[j4], add=True)

    def scat_wait(j8, j4):
        pltpu.make_async_copy(rows[j4], acc_sh.at[didx[j8]], sems[j4]).wait()

    def stage(b, j8, head=False, tail=0):
        j4 = j8 % 4
        if not head:
            scat_wait((j8 + 6) % 8, (j4 + 2) % 4)
        if tail < 1:
            idx_issue(b + 4, (j8 + 4) % 8)
        if tail < 2:
            idx_wait(b + 2, (j8 + 2) % 8)
            gat_issue((j8 + 2) % 8, (j4 + 2) % 4)
        gat_wait(j8, j4)
        scat_issue(j8, j4)

    idx_sync(0, 0)
    idx_sync(1, 1)
    idx_issue(2, 2)
    idx_issue(3, 3)
    gat_issue(0, 0)
    gat_issue(1, 1)
    plsc.subcore_barrier()

    stage(0, 0, head=True)
    stage(1, 1, head=True)
    for b in range(2, 8):
        stage(b, b % 8)

    def body(i, carry):
        b0 = 8 + i * 8
        for j in range(8):
            stage(b0 + j, j)
        return carry

    lax.fori_loop(0, (NBLK - 13) // 8, body, 0)

    stage(NBLK - 5, 0)
    stage(NBLK - 4, 1, tail=1)
    stage(NBLK - 3, 2, tail=1)
    stage(NBLK - 2, 3, tail=2)
    stage(NBLK - 1, 4, tail=2)
    scat_wait(3, 3)
    scat_wait(4, 0)
    plsc.subcore_barrier()
    pltpu.sync_copy(
        acc_sh.at[pl.ds(sid * RPT, RPT)], out_hbm.at[cid, pl.ds(sid * RPT, RPT)]
    )


def _agg_call(src1, dst1, mt, z128):
    idx_t = pltpu.VMEM((BLK,), jnp.int32)
    row_t = pltpu.VMEM((BLK, HID), jnp.float32)
    dma = pltpu.SemaphoreType.DMA
    return pl.kernel(
        _agg_body,
        out_type=jax.ShapeDtypeStruct((NC, N_PAD, HID), jnp.float32),
        mesh=_mesh(),
        scratch_types=(
            [idx_t] * 16 + [row_t] * 4
            + [pltpu.VMEM_SHARED((N_PAD, HID), jnp.float32)]
            + [dma] * 16
        ),
    )(src1, dst1, mt, z128)


_BNS = float(1.0 / (1.0 + EPS) ** 0.5)


def _mm_t(a, w):
    return lax.dot_general(a, w, (((1,), (1,)), ((), ())),
                           preferred_element_type=jnp.float32)


def _silu(x):
    return x * jax.nn.sigmoid(x)


def _tc0a_body(x_ref, pp_ref, wx_ref, wp_ref, bin_ref, gin_ref, bein_ref,
               h_ref):
    lin = _mm_t(x_ref[...], wx_ref[...]) + _mm_t(pp_ref[...], wp_ref[...])
    lin = lin + bin_ref[...]
    y = lin * (gin_ref[...] * _BNS) + bein_ref[...]
    h_ref[...] = _silu(y)


def _tc0a_call(x, pos, Wx, Wp, b_in, g_in, be_in):
    rb = lambda i: (i, 0)
    wb = lambda i: (0, 0)
    return pl.pallas_call(
        _tc0a_body,
        grid=(GRID,),
        in_specs=[
            pl.BlockSpec((ROWS, D_FEAT), rb),
            pl.BlockSpec((ROWS, 3), rb),
            pl.BlockSpec((HID, D_FEAT), wb),
            pl.BlockSpec((HID, 3), wb),
            pl.BlockSpec((1, HID), wb),
            pl.BlockSpec((1, HID), wb),
            pl.BlockSpec((1, HID), wb),
        ],
        out_specs=pl.BlockSpec((ROWS, HID), rb),
        out_shape=jax.ShapeDtypeStruct((N, HID), jnp.float32),
    )(x, pos, Wx, Wp, b_in, g_in, be_in)


def _tc0b_body(h_ref, dp_ref, wg0_ref, mt_ref, dinv_ref):
    deg = dp_ref[0, :, 0:1] + dp_ref[1, :, 0:1] + 1.0
    dinv = lax.rsqrt(deg)
    dinv_ref[...] = jnp.broadcast_to(dinv, (ROWS, 8))
    mt_ref[...] = _mm_t(h_ref[...], wg0_ref[...]) * dinv


def _tc0b_call(h, dp8, Wg0):
    rb = lambda i: (i, 0)
    return pl.pallas_call(
        _tc0b_body,
        grid=(GRID,),
        in_specs=[
            pl.BlockSpec((ROWS, HID), rb),
            pl.BlockSpec((NC, ROWS, HID), lambda i: (0, i, 0)),
            pl.BlockSpec((HID, HID), lambda i: (0, 0)),
        ],
        out_specs=[
            pl.BlockSpec((ROWS, HID), rb),
            pl.BlockSpec((ROWS, 8), rb),
        ],
        out_shape=[
            jax.ShapeDtypeStruct((N, HID), jnp.float32),
            jax.ShapeDtypeStruct((N, 8), jnp.float32),
        ],
    )(h, dp8, Wg0)


def _epi_body(p_ref, mt_ref, hp_ref, dinv_ref, bg_ref, gn_ref, bn_ref,
              wgn_ref, h_ref, mtn_ref):
    dinv = jnp.broadcast_to(dinv_ref[:, 0:1], (ROWS, HID))
    agg = p_ref[0] + p_ref[1]
    out = dinv * (agg + mt_ref[...]) + bg_ref[...]
    y = out * (gn_ref[...] * _BNS) + bn_ref[...]
    h = _silu(y + hp_ref[...])
    h_ref[...] = h
    mtn_ref[...] = _mm_t(h, wgn_ref[...]) * dinv


def _epi_call(p, mt, h_prev, dinv8, bg, gn, bn, Wg_next):
    rb = lambda i: (i, 0)
    wb = lambda i: (0, 0)
    return pl.pallas_call(
        _epi_body,
        grid=(GRID,),
        in_specs=[
            pl.BlockSpec((NC, ROWS, HID), lambda i: (0, i, 0)),
            pl.BlockSpec((ROWS, HID), rb),
            pl.BlockSpec((ROWS, HID), rb),
            pl.BlockSpec((ROWS, 8), rb),
            pl.BlockSpec((1, HID), wb),
            pl.BlockSpec((1, HID), wb),
            pl.BlockSpec((1, HID), wb),
            pl.BlockSpec((HID, HID), wb),
        ],
        out_specs=[
            pl.BlockSpec((ROWS, HID), rb),
            pl.BlockSpec((ROWS, HID), rb),
        ],
        out_shape=[
            jax.ShapeDtypeStruct((N, HID), jnp.float32),
            jax.ShapeDtypeStruct((N, HID), jnp.float32),
        ],
    )(p, mt, h_prev, dinv8, bg, gn, bn, Wg_next)


def _epi_pool_body(p_ref, mt_ref, hp_ref, dinv_ref, bg_ref, gn_ref, bn_ref,
                   watt_ref, wo1_ref, bo1_ref, wo2_ref, bo2_ref, wo3_ref,
                   h_ref, pred_ref, m_sc, s_sc, v_sc):
    i = pl.program_id(0)
    dinv = jnp.broadcast_to(dinv_ref[:, 0:1], (ROWS, HID))
    agg = p_ref[0] + p_ref[1]
    out = dinv * (agg + mt_ref[...]) + bg_ref[...]
    y = out * (gn_ref[...] * _BNS) + bn_ref[...]
    h = _silu(y + hp_ref[...])
    h_ref[...] = h
    wl = jnp.sum(h * watt_ref[...], axis=1, keepdims=True)
    mb = jnp.max(wl)

    @pl.when(i == 0)
    def _():
        m_sc[0, 0] = mb
        s_sc[0, 0] = 0.0
        v_sc[...] = jnp.zeros((1, HID), jnp.float32)

    m_old = m_sc[0, 0]
    m_new = jnp.maximum(m_old, mb)
    corr = jnp.exp(m_old - m_new)
    e = jnp.exp(wl - m_new)
    s_sc[0, 0] = s_sc[0, 0] * corr + jnp.sum(e)
    v_sc[...] = v_sc[...] * corr + jnp.sum(h * e, axis=0, keepdims=True)
    m_sc[0, 0] = m_new

    @pl.when(i == GRID - 1)
    def _():
        hg = v_sc[...] / s_sc[0, 0]
        o = _silu(_mm_t(hg, wo1_ref[...]) + bo1_ref[...])
        o = _silu(_mm_t(o, wo2_ref[...]) + bo2_ref[...])
        pred_ref[...] = _mm_t(o, wo3_ref[...])


def _epi_pool_call(p, mt, h_prev, dinv8, bg, gn, bn, W_att, Wo1, bo1, Wo2,
                   bo2, Wo3):
    rb = lambda i: (i, 0)
    wb = lambda i: (0, 0)
    return pl.pallas_call(
        _epi_pool_body,
        grid=(GRID,),
        in_specs=[
            pl.BlockSpec((NC, ROWS, HID), lambda i: (0, i, 0)),
            pl.BlockSpec((ROWS, HID), rb),
            pl.BlockSpec((ROWS, HID), rb),
            pl.BlockSpec((ROWS, 8), rb),
            pl.BlockSpec((1, HID), wb),
            pl.BlockSpec((1, HID), wb),
            pl.BlockSpec((1, HID), wb),
            pl.BlockSpec((1, HID), wb),
            pl.BlockSpec((HID, HID), wb),
            pl.BlockSpec((1, HID), wb),
            pl.BlockSpec((HID // 2, HID), wb),
            pl.BlockSpec((1, HID // 2), wb),
            pl.BlockSpec((1, HID // 2), wb),
        ],
        out_specs=[
            pl.BlockSpec((ROWS, HID), rb),
            pl.BlockSpec((1, 1), wb),
        ],
        out_shape=[
            jax.ShapeDtypeStruct((N, HID), jnp.float32),
            jax.ShapeDtypeStruct((1, 1), jnp.float32),
        ],
        scratch_shapes=[
            pltpu.SMEM((1, 1), jnp.float32),
            pltpu.SMEM((1, 1), jnp.float32),
            pltpu.VMEM((1, HID), jnp.float32),
        ],
    )(p, mt, h_prev, dinv8, bg, gn, bn, W_att, Wo1, bo1, Wo2, bo2, Wo3)


def kernel(x, pos, edge_index, W_in, b_in, g_in, be_in, Wg0, bg0, gn0, bn0,
           Wg1, bg1, gn1, bn1, Wg2, bg2, gn2, bn2, Wg3, bg3, gn3, bn3,
           W_att, b_att, Wo1, bo1, Wo2, bo2, Wo3, bo3):
    f32 = jnp.float32
    src1 = edge_index[0]
    dst1 = edge_index[1]
    Wx = W_in[:, :D_FEAT]
    Wp = W_in[:, D_FEAT:]
    z128 = jnp.zeros((RPT, HID), f32)
    ones_blk = jnp.ones((BLK, HID), f32)

    dp = _deg_call(dst1, ones_blk, z128)
    h = _tc0a_call(x, pos, Wx, Wp, b_in.reshape(1, HID),
                   g_in.reshape(1, HID), be_in.reshape(1, HID))
    mt, dinv8 = _tc0b_call(h, dp, Wg0)

    mids = [(bg0, gn0, bn0, Wg1), (bg1, gn1, bn1, Wg2), (bg2, gn2, bn2, Wg3)]
    for bg, gn, bn, Wg_next in mids:
        p = _agg_call(src1, dst1, mt, z128)
        h, mt = _epi_call(p, mt, h, dinv8, bg.reshape(1, HID),
                          gn.reshape(1, HID), bn.reshape(1, HID), Wg_next)
    p = _agg_call(src1, dst1, mt, z128)
    h, pred = _epi_pool_call(p, mt, h, dinv8, bg3.reshape(1, HID),
                             gn3.reshape(1, HID), bn3.reshape(1, HID),
                             W_att, Wo1, bo1.reshape(1, HID), Wo2,
                             bo2.reshape(1, HID // 2), Wo3)
    pred = pred + bo3.reshape(1, 1)
    return (pred, h)


# speedup vs baseline: 1.0270x; 1.0165x over previous
import jax
import jax.numpy as jnp
from jax import lax
from jax.experimental import pallas as pl
from jax.experimental.pallas import tpu as pltpu
from jax.experimental.pallas import tpu_sc as plsc

N = 10000
E = 320000
D_FEAT = 128
HID = 128
EPS = 1e-5

NC = 2
NS = 16
NW = NC * NS
EPW = E // NW
BLK = 80
NBLK = EPW // BLK
N_PAD = 10240
RPT = N_PAD // NS

ROWS = 2000
GRID = N // ROWS

_mesh_cache = []


def _mesh():
    if not _mesh_cache:
        _mesh_cache.append(plsc.VectorSubcoreMesh(
            core_axis_name="c", subcore_axis_name="s",
            num_cores=NC, num_subcores=NS))
    return _mesh_cache[0]


def _deg_body(dst_hbm, ones_hbm, z_hbm, out_hbm, ones_v, didx0, didx1,
              acc_sh, sem_a, sem_b, semi0, semi1):
    cid = lax.axis_index("c")
    sid = lax.axis_index("s")
    wid = sid * NC + cid
    pltpu.sync_copy(z_hbm, acc_sh.at[pl.ds(sid * RPT, RPT)])
    pltpu.sync_copy(ones_hbm, ones_v)
    base = wid * EPW

    def idx_issue(b, dbuf, sem):
        off = pl.multiple_of(base + b * BLK, 8)
        pltpu.async_copy(dst_hbm.at[pl.ds(off, BLK)], dbuf, sem)

    def idx_wait(b, dbuf, sem):
        off = pl.multiple_of(base + b * BLK, 8)
        pltpu.make_async_copy(dst_hbm.at[pl.ds(off, BLK)], dbuf, sem).wait()

    def scat(dbuf, sem):
        pltpu.async_copy(ones_v, acc_sh.at[dbuf], sem, add=True)

    def scat_wait(dbuf, sem):
        pltpu.make_async_copy(ones_v, acc_sh.at[dbuf], sem).wait()

    off0 = pl.multiple_of(base, 8)
    pltpu.sync_copy(dst_hbm.at[pl.ds(off0, BLK)], didx0)
    idx_issue(1, didx1, semi1)
    plsc.subcore_barrier()
    scat(didx0, sem_a)

    def body(i, carry):
        b0 = i * 2
        idx_wait(b0 + 1, didx1, semi1)
        scat_wait(didx0, sem_a)
        scat(didx1, sem_b)
        idx_issue(b0 + 2, didx0, semi0)
        idx_wait(b0 + 2, didx0, semi0)
        scat_wait(didx1, sem_b)
        scat(didx0, sem_a)
        b3 = jnp.minimum(b0 + 3, NBLK - 1)
        idx_issue(b3, didx1, semi1)
        return carry

    lax.fori_loop(0, (NBLK - 1) // 2, body, 0)
    scat_wait(didx0, sem_a)
    idx_wait(NBLK - 1, didx1, semi1)
    plsc.subcore_barrier()
    pltpu.sync_copy(
        acc_sh.at[pl.ds(sid * RPT, RPT)], out_hbm.at[cid, pl.ds(sid * RPT, RPT)]
    )


def _deg_call(dst, ones_blk, z128):
    return pl.kernel(
        _deg_body,
        out_type=jax.ShapeDtypeStruct((NC, N_PAD, HID), jnp.float32),
        mesh=_mesh(),
        scratch_types=[
            pltpu.VMEM((BLK, HID), jnp.float32),
            pltpu.VMEM((BLK,), jnp.int32),
            pltpu.VMEM((BLK,), jnp.int32),
            pltpu.VMEM_SHARED((N_PAD, HID), jnp.float32),
            pltpu.SemaphoreType.DMA,
            pltpu.SemaphoreType.DMA,
            pltpu.SemaphoreType.DMA,
            pltpu.SemaphoreType.DMA,
        ],
    )(dst, ones_blk, z128)


def _agg_body(src_hbm, dst_hbm, mt_hbm, z_hbm, out_hbm, *scr):
    sidx = scr[0:8]
    didx = scr[8:16]
    rows = scr[16:20]
    acc_sh = scr[20]
    semg = scr[21:25]
    sems = scr[25:29]
    semi = scr[29:37]
    cid = lax.axis_index("c")
    sid = lax.axis_index("s")
    wid = sid * NC + cid
    pltpu.sync_copy(z_hbm, acc_sh.at[pl.ds(sid * RPT, RPT)])
    base = wid * EPW

    def idx_issue(b, j8):
        off = pl.multiple_of(base + b * BLK, 8)
        pltpu.async_copy(src_hbm.at[pl.ds(off, BLK)], sidx[j8], semi[j8])
        pltpu.async_copy(dst_hbm.at[pl.ds(off, BLK)], didx[j8], semi[j8])

    def idx_wait(b, j8):
        off = pl.multiple_of(base + b * BLK, 8)
        pltpu.make_async_copy(src_hbm.at[pl.ds(off, BLK)], sidx[j8], semi[j8]).wait()
        pltpu.make_async_copy(dst_hbm.at[pl.ds(off, BLK)], didx[j8], semi[j8]).wait()

    def idx_sync(b, j8):
        off = pl.multiple_of(base + b * BLK, 8)
        pltpu.sync_copy(src_hbm.at[pl.ds(off, BLK)], sidx[j8])
        pltpu.sync_copy(dst_hbm.at[pl.ds(off, BLK)], didx[j8])

    def gat_issue(j8, j4):
        pltpu.async_copy(mt_hbm.at[sidx[j8]], rows[j4], semg[j4])

    def gat_wait(j8, j4):
        pltpu.make_async_copy(mt_hbm.at[sidx[j8]], rows[j4], semg[j4]).wait()

    def scat_issue(j8, j4):
        pltpu.async_copy(rows[j4], acc_sh.at[didx[j8]], sems[j4], add=True)

    def scat_wait(j8, j4):
        pltpu.make_async_copy(rows[j4], acc_sh.at[didx[j8]], sems[j4]).wait()

    def stage(b, j8, head=False, tail=0):
        j4 = j8 % 4
        if not head:
            scat_wait((j8 + 6) % 8, (j4 + 2) % 4)
        if tail < 1:
            idx_issue(b + 4, (j8 + 4) % 8)
        if tail < 2:
            idx_wait(b + 2, (j8 + 2) % 8)
            gat_issue((j8 + 2) % 8, (j4 + 2) % 4)
        gat_wait(j8, j4)
        scat_issue(j8, j4)

    idx_sync(0, 0)
    idx_sync(1, 1)
    idx_issue(2, 2)
    idx_issue(3, 3)
    gat_issue(0, 0)
    gat_issue(1, 1)
    plsc.subcore_barrier()

    stage(0, 0, head=True)
    stage(1, 1, head=True)
    for b in range(2, 8):
        stage(b, b % 8)

    def body(i, carry):
        b0 = 8 + i * 8
        for j in range(8):
            stage(b0 + j, j)
        return carry

    lax.fori_loop(0, (NBLK - 13) // 8, body, 0)

    stage(NBLK - 5, 0)
    stage(NBLK - 4, 1, tail=1)
    stage(NBLK - 3, 2, tail=1)
    stage(NBLK - 2, 3, tail=2)
    stage(NBLK - 1, 4, tail=2)
    scat_wait(3, 3)
    scat_wait(4, 0)
    plsc.subcore_barrier()
    pltpu.sync_copy(
        acc_sh.at[pl.ds(sid * RPT, RPT)], out_hbm.at[cid, pl.ds(sid * RPT, RPT)]
    )


def _agg_call(src1, dst1, mt, z128):
    idx_t = pltpu.VMEM((BLK,), jnp.int32)
    row_t = pltpu.VMEM((BLK, HID), jnp.float32)
    dma = pltpu.SemaphoreType.DMA
    return pl.kernel(
        _agg_body,
        out_type=jax.ShapeDtypeStruct((NC, N_PAD, HID), jnp.float32),
        mesh=_mesh(),
        scratch_types=(
            [idx_t] * 16 + [row_t] * 4
            + [pltpu.VMEM_SHARED((N_PAD, HID), jnp.float32)]
            + [dma] * 16
        ),
    )(src1, dst1, mt, z128)


_BNS = float(1.0 / (1.0 + EPS) ** 0.5)


def _mm_t(a, w):
    return lax.dot_general(a, w, (((1,), (1,)), ((), ())),
                           preferred_element_type=jnp.float32)


def _silu(x):
    return x * jax.nn.sigmoid(x)


def _tc0a_body(x_ref, pp_ref, wx_ref, wp_ref, bin_ref, gin_ref, bein_ref,
               h_ref):
    lin = _mm_t(x_ref[...], wx_ref[...]) + _mm_t(pp_ref[...], wp_ref[...])
    lin = lin + bin_ref[...]
    y = lin * (gin_ref[...] * _BNS) + bein_ref[...]
    h_ref[...] = _silu(y)


def _tc0a_call(x, pos, Wx, Wp, b_in, g_in, be_in):
    rb = lambda i: (i, 0)
    wb = lambda i: (0, 0)
    return pl.pallas_call(
        _tc0a_body,
        grid=(GRID,),
        in_specs=[
            pl.BlockSpec((ROWS, D_FEAT), rb),
            pl.BlockSpec((ROWS, 3), rb),
            pl.BlockSpec((HID, D_FEAT), wb),
            pl.BlockSpec((HID, 3), wb),
            pl.BlockSpec((1, HID), wb),
            pl.BlockSpec((1, HID), wb),
            pl.BlockSpec((1, HID), wb),
        ],
        out_specs=pl.BlockSpec((ROWS, HID), rb),
        out_shape=jax.ShapeDtypeStruct((N, HID), jnp.float32),
    )(x, pos, Wx, Wp, b_in, g_in, be_in)


def _tc0b_body(h_ref, dp_ref, wg0_ref, mt_ref, dinv_ref):
    deg = dp_ref[0, :, 0:1] + dp_ref[1, :, 0:1] + 1.0
    dinv = lax.rsqrt(deg)
    dinv_ref[...] = jnp.broadcast_to(dinv, (ROWS, 8))
    mt_ref[...] = _mm_t(h_ref[...], wg0_ref[...]) * dinv


def _tc0b_call(h, dp8, Wg0):
    rb = lambda i: (i, 0)
    return pl.pallas_call(
        _tc0b_body,
        grid=(GRID,),
        in_specs=[
            pl.BlockSpec((ROWS, HID), rb),
            pl.BlockSpec((NC, ROWS, HID), lambda i: (0, i, 0)),
            pl.BlockSpec((HID, HID), lambda i: (0, 0)),
        ],
        out_specs=[
            pl.BlockSpec((ROWS, HID), rb),
            pl.BlockSpec((ROWS, 8), rb),
        ],
        out_shape=[
            jax.ShapeDtypeStruct((N, HID), jnp.float32),
            jax.ShapeDtypeStruct((N, 8), jnp.float32),
        ],
    )(h, dp8, Wg0)


def _epi_body(p_ref, mt_ref, hp_ref, dinv_ref, bg_ref, gn_ref, bn_ref,
              wgn_ref, h_ref, mtn_ref):
    dinv = jnp.broadcast_to(dinv_ref[:, 0:1], (ROWS, HID))
    agg = p_ref[0] + p_ref[1]
    out = dinv * (agg + mt_ref[...]) + bg_ref[...]
    y = out * (gn_ref[...] * _BNS) + bn_ref[...]
    h = _silu(y + hp_ref[...])
    h_ref[...] = h
    mtn_ref[...] = _mm_t(h, wgn_ref[...]) * dinv


def _epi_call(p, mt, h_prev, dinv8, bg, gn, bn, Wg_next):
    rb = lambda i: (i, 0)
    wb = lambda i: (0, 0)
    return pl.pallas_call(
        _epi_body,
        grid=(GRID,),
        in_specs=[
            pl.BlockSpec((NC, ROWS, HID), lambda i: (0, i, 0)),
            pl.BlockSpec((ROWS, HID), rb),
            pl.BlockSpec((ROWS, HID), rb),
            pl.BlockSpec((ROWS, 8), rb),
            pl.BlockSpec((1, HID), wb),
            pl.BlockSpec((1, HID), wb),
            pl.BlockSpec((1, HID), wb),
            pl.BlockSpec((HID, HID), wb),
        ],
        out_specs=[
            pl.BlockSpec((ROWS, HID), rb),
            pl.BlockSpec((ROWS, HID), rb),
        ],
        out_shape=[
            jax.ShapeDtypeStruct((N, HID), jnp.float32),
            jax.ShapeDtypeStruct((N, HID), jnp.float32),
        ],
    )(p, mt, h_prev, dinv8, bg, gn, bn, Wg_next)


def _epi_pool_body(p_ref, mt_ref, hp_ref, dinv_ref, bg_ref, gn_ref, bn_ref,
                   watt_ref, wo1_ref, bo1_ref, wo2_ref, bo2_ref, wo3_ref,
                   h_ref, pred_ref, m_sc, s_sc, v_sc):
    i = pl.program_id(0)
    dinv = jnp.broadcast_to(dinv_ref[:, 0:1], (ROWS, HID))
    agg = p_ref[0] + p_ref[1]
    out = dinv * (agg + mt_ref[...]) + bg_ref[...]
    y = out * (gn_ref[...] * _BNS) + bn_ref[...]
    h = _silu(y + hp_ref[...])
    h_ref[...] = h
    wl = jnp.sum(h * watt_ref[...], axis=1, keepdims=True)
    mb = jnp.max(wl)

    @pl.when(i == 0)
    def _():
        m_sc[0, 0] = mb
        s_sc[0, 0] = 0.0
        v_sc[...] = jnp.zeros((1, HID), jnp.float32)

    m_old = m_sc[0, 0]
    m_new = jnp.maximum(m_old, mb)
    corr = jnp.exp(m_old - m_new)
    e = jnp.exp(wl - m_new)
    s_sc[0, 0] = s_sc[0, 0] * corr + jnp.sum(e)
    v_sc[...] = v_sc[...] * corr + jnp.sum(h * e, axis=0, keepdims=True)
    m_sc[0, 0] = m_new

    @pl.when(i == GRID - 1)
    def _():
        hg = v_sc[...] / s_sc[0, 0]
        o = _silu(_mm_t(hg, wo1_ref[...]) + bo1_ref[...])
        o = _silu(_mm_t(o, wo2_ref[...]) + bo2_ref[...])
        pred_ref[...] = _mm_t(o, wo3_ref[...])


def _epi_pool_call(p, mt, h_prev, dinv8, bg, gn, bn, W_att, Wo1, bo1, Wo2,
                   bo2, Wo3):
    rb = lambda i: (i, 0)
    wb = lambda i: (0, 0)
    return pl.pallas_call(
        _epi_pool_body,
        grid=(GRID,),
        in_specs=[
            pl.BlockSpec((NC, ROWS, HID), lambda i: (0, i, 0)),
            pl.BlockSpec((ROWS, HID), rb),
            pl.BlockSpec((ROWS, HID), rb),
            pl.BlockSpec((ROWS, 8), rb),
            pl.BlockSpec((1, HID), wb),
            pl.BlockSpec((1, HID), wb),
            pl.BlockSpec((1, HID), wb),
            pl.BlockSpec((1, HID), wb),
            pl.BlockSpec((HID, HID), wb),
            pl.BlockSpec((1, HID), wb),
            pl.BlockSpec((HID // 2, HID), wb),
            pl.BlockSpec((1, HID // 2), wb),
            pl.BlockSpec((1, HID // 2), wb),
        ],
        out_specs=[
            pl.BlockSpec((ROWS, HID), rb),
            pl.BlockSpec((1, 1), wb),
        ],
        out_shape=[
            jax.ShapeDtypeStruct((N, HID), jnp.float32),
            jax.ShapeDtypeStruct((1, 1), jnp.float32),
        ],
        scratch_shapes=[
            pltpu.SMEM((1, 1), jnp.float32),
            pltpu.SMEM((1, 1), jnp.float32),
            pltpu.VMEM((1, HID), jnp.float32),
        ],
    )(p, mt, h_prev, dinv8, bg, gn, bn, W_att, Wo1, bo1, Wo2, bo2, Wo3)


def kernel(x, pos, edge_index, W_in, b_in, g_in, be_in, Wg0, bg0, gn0, bn0,
           Wg1, bg1, gn1, bn1, Wg2, bg2, gn2, bn2, Wg3, bg3, gn3, bn3,
           W_att, b_att, Wo1, bo1, Wo2, bo2, Wo3, bo3):
    f32 = jnp.float32
    src1 = edge_index[0]
    dst1 = edge_index[1]
    Wx = W_in[:, :D_FEAT]
    Wp = W_in[:, D_FEAT:]
    z128 = jnp.zeros((RPT, HID), f32)
    ones_blk = jnp.ones((BLK, HID), f32)

    dp = _deg_call(dst1, ones_blk, z128)
    h = _tc0a_call(x, pos, Wx, Wp, b_in.reshape(1, HID),
                   g_in.reshape(1, HID), be_in.reshape(1, HID))
    mt, dinv8 = _tc0b_call(h, dp, Wg0)

    mids = [(bg0, gn0, bn0, Wg1), (bg1, gn1, bn1, Wg2), (bg2, gn2, bn2, Wg3)]
    for bg, gn, bn, Wg_next in mids:
        p = _agg_call(src1, dst1, mt, z128)
        h, mt = _epi_call(p, mt, h, dinv8, bg.reshape(1, HID),
                          gn.reshape(1, HID), bn.reshape(1, HID), Wg_next)
    p = _agg_call(src1, dst1, mt, z128)
    h, pred = _epi_pool_call(p, mt, h, dinv8, bg3.reshape(1, HID),
                             gn3.reshape(1, HID), bn3.reshape(1, HID),
                             W_att, Wo1, bo1.reshape(1, HID), Wo2,
                             bo2.reshape(1, HID // 2), Wo3)
    pred = pred + bo3.reshape(1, 1)
    return (pred, h)
